# baseline (speedup 1.0000x reference)
import jax
import jax.numpy as jnp
from jax import lax
from jax.experimental import pallas as pl
from jax.experimental.pallas import tpu as pltpu
from jax.experimental.pallas import tpu_sc as plsc

N_ROWS = 1_000_000
D = 32
SCALE = D ** 0.5

NC = 2
NS = 16
NW = NC * NS

B_SAMPLES = 16384
SEQ = 50
S_PER_W = B_SAMPLES // NW
IDX_PER_W = S_PER_W * SEQ
GROUP = 16
ROWS_PER_GROUP = GROUP * SEQ
GROUPS = S_PER_W // GROUP
DESCS = ((0, 128), (128, 128), (256, 128), (384, 128),
         (512, 128), (640, 128), (768, 32))

MUL_UNROLL = 10


_T_BLK = 16384


def _scale_body(x_ref, o_ref, p_ref):
    x = x_ref[...] * SCALE
    o_ref[...] = x
    xt = x.T
    p_ref[...] = jnp.concatenate(
        [xt, jnp.zeros((_T_BLK, 128 - D), jnp.float32)], axis=1)


def _scale_table(weight):
    wt = weight.T
    grid = (N_ROWS + _T_BLK - 1) // _T_BLK
    mt, mpad = pl.pallas_call(
        _scale_body,
        out_shape=(
            jax.ShapeDtypeStruct((D, N_ROWS), jnp.float32),
            jax.ShapeDtypeStruct((N_ROWS, 128), jnp.float32),
        ),
        grid=(grid,),
        in_specs=[pl.BlockSpec((D, _T_BLK), lambda i: (0, i))],
        out_specs=(
            pl.BlockSpec((D, _T_BLK), lambda i: (0, i)),
            pl.BlockSpec((_T_BLK, 128), lambda i: (i, 0)),
        ),
    )(wt)
    return mt.T, mpad


def _transpose_relayout(ra, rb, obuf):
    base = lax.iota(jnp.int32, 16) * SEQ
    for half, src in ((0, ra), (1, rb)):
        @pl.loop(0, 16)
        def _f(f):
            col = jnp.zeros((16,), jnp.int32) + f
            @pl.loop(0, SEQ, unroll=MUL_UNROLL)
            def _r(r):
                rows = base + r
                obuf[r, half * 16 + f, :] = plsc.load_gather(src, [rows, col])


def _gather_body(idx_hbm, w_hbm, lu_hbm,
                 idx_v, a0, b0, a1, b1, la0, lb0, la1, lb1, obuf,
                 g_in0, g_in1, g_osem):
    c = lax.axis_index("c")
    s = lax.axis_index("s")
    wid = s * NC + c

    pltpu.sync_copy(idx_hbm.at[wid], idx_v)
    s_base = wid * S_PER_W
    rabufs = ((a0, b0), (a1, b1))
    lists = ((la0, lb0), (la1, lb1))
    g_isems = (g_in0, g_in1)

    def build_lists(g, b):
        la, lb = lists[b]
        @pl.loop(0, ROWS_PER_GROUP // 16, unroll=5)
        def _k(k):
            v = idx_v[pl.ds(g * ROWS_PER_GROUP + k * 16, 16)] * 8
            la[pl.ds(k * 16, 16)] = v
            lb[pl.ds(k * 16, 16)] = v + 1

    def fire_group(g, b):
        ra, rb = rabufs[b]
        la, lb = lists[b]
        for off, n in DESCS:
            pltpu.async_copy(w_hbm.at[la.at[pl.ds(off, n)]],
                             ra.at[pl.ds(off, n)], g_isems[b])
            pltpu.async_copy(w_hbm.at[lb.at[pl.ds(off, n)]],
                             rb.at[pl.ds(off, n)], g_isems[b])

    def wait_group(g, b):
        ra, rb = rabufs[b]
        la, lb = lists[b]
        for off, n in DESCS:
            pltpu.make_async_copy(w_hbm.at[la.at[pl.ds(off, n)]],
                                  ra.at[pl.ds(off, n)], g_isems[b]).wait()
            pltpu.make_async_copy(w_hbm.at[lb.at[pl.ds(off, n)]],
                                  rb.at[pl.ds(off, n)], g_isems[b]).wait()

    def out_dst(g):
        return lu_hbm.at[:, :, pl.ds(s_base + g * GROUP, GROUP)]

    build_lists(0, 0)
    fire_group(0, 0)

    @pl.loop(0, GROUPS // 2)
    def _g(p):
        for b in range(2):
            g = 2 * p + b
            nb = 1 - b
            wait_group(g, b)
            @pl.when(g + 1 < GROUPS)
            def _():
                build_lists(g + 1, nb)
                fire_group(g + 1, nb)
            @pl.when(g >= 1)
            def _():
                pltpu.make_async_copy(obuf, out_dst(g - 1), g_osem).wait()
            _transpose_relayout(rabufs[b][0], rabufs[b][1], obuf)
            pltpu.async_copy(obuf, out_dst(g), g_osem)

    pltpu.make_async_copy(obuf, out_dst(GROUPS - 1), g_osem).wait()


def _sc_gather(idx_flat, w16):
    mesh = plsc.VectorSubcoreMesh(core_axis_name="c", subcore_axis_name="s")
    k = pl.kernel(
        _gather_body,
        out_type=jax.ShapeDtypeStruct((SEQ, D, B_SAMPLES), jnp.float32),
        mesh=mesh,
        compiler_params=pltpu.CompilerParams(use_tc_tiling_on_sc=False,
                                             needs_layout_passes=False),
        scratch_types=[
            pltpu.VMEM((IDX_PER_W,), jnp.int32),
            pltpu.VMEM((ROWS_PER_GROUP, 16), jnp.float32),
            pltpu.VMEM((ROWS_PER_GROUP, 16), jnp.float32),
            pltpu.VMEM((ROWS_PER_GROUP, 16), jnp.float32),
            pltpu.VMEM((ROWS_PER_GROUP, 16), jnp.float32),
            pltpu.VMEM((ROWS_PER_GROUP,), jnp.int32),
            pltpu.VMEM((ROWS_PER_GROUP,), jnp.int32),
            pltpu.VMEM((ROWS_PER_GROUP,), jnp.int32),
            pltpu.VMEM((ROWS_PER_GROUP,), jnp.int32),
            pltpu.VMEM((SEQ, D, GROUP), jnp.float32),
        ] + [pltpu.SemaphoreType.DMA] * 3,
    )
    return k(idx_flat, w16)


def kernel(indices, weight):
    masked, mpad = _scale_table(weight)
    w16 = mpad.reshape(N_ROWS * 8, 16)
    idx_flat = indices.astype(jnp.int32).reshape(NW, IDX_PER_W)
    lu_t = _sc_gather(idx_flat, w16)
    return lu_t.transpose(2, 0, 1), masked


# speedup vs baseline: 1.1474x; 1.1474x over previous
import jax
import jax.numpy as jnp
from jax import lax
from jax.experimental import pallas as pl
from jax.experimental.pallas import tpu as pltpu
from jax.experimental.pallas import tpu_sc as plsc

N_ROWS = 1_000_000
D = 32
SCALE = D ** 0.5

NC = 2
NS = 16
NW = NC * NS

B_SAMPLES = 16384
SEQ = 50
S_PER_W = B_SAMPLES // NW
IDX_PER_W = S_PER_W * SEQ
GROUP = 16
ROWS_PER_GROUP = GROUP * SEQ
GROUPS = S_PER_W // GROUP
DESCS = ((0, 128), (128, 128), (256, 128), (384, 128),
         (512, 128), (640, 128), (768, 32))

MUL_UNROLL = 10


_T_BLK = 16384


def _scale_body(x_ref, o_ref, p_ref):
    x = x_ref[...] * SCALE
    o_ref[...] = x
    xt = x.T
    p_ref[...] = jnp.concatenate(
        [xt, jnp.zeros((_T_BLK, 128 - D), jnp.float32)], axis=1)


def _scale_table(weight):
    wt = weight.T
    grid = (N_ROWS + _T_BLK - 1) // _T_BLK
    mt, mpad = pl.pallas_call(
        _scale_body,
        out_shape=(
            jax.ShapeDtypeStruct((D, N_ROWS), jnp.float32),
            jax.ShapeDtypeStruct((N_ROWS, 128), jnp.float32),
        ),
        grid=(grid,),
        in_specs=[pl.BlockSpec((D, _T_BLK), lambda i: (0, i))],
        out_specs=(
            pl.BlockSpec((D, _T_BLK), lambda i: (0, i)),
            pl.BlockSpec((_T_BLK, 128), lambda i: (i, 0)),
        ),
    )(wt)
    return mt.T, mpad


def _scale_relayout(ra, rb, obuf):
    @pl.loop(0, GROUP)
    def _s(si):
        @pl.loop(0, SEQ, unroll=MUL_UNROLL)
        def _row(r):
            flat = si * SEQ + r
            obuf[si, r, pl.ds(0, 16)] = ra[flat, :]
            obuf[si, r, pl.ds(16, 16)] = rb[flat, :]


def _gather_body(idx_hbm, w_hbm, lu_hbm,
                 idx_v, a0, b0, a1, b1, la0, lb0, la1, lb1, obuf,
                 g_in0, g_in1, g_osem):
    c = lax.axis_index("c")
    s = lax.axis_index("s")
    wid = s * NC + c

    pltpu.sync_copy(idx_hbm.at[wid], idx_v)
    s_base = wid * S_PER_W
    rabufs = ((a0, b0), (a1, b1))
    lists = ((la0, lb0), (la1, lb1))
    g_isems = (g_in0, g_in1)

    def build_lists(g, b):
        la, lb = lists[b]
        @pl.loop(0, ROWS_PER_GROUP // 16, unroll=5)
        def _k(k):
            v = idx_v[pl.ds(g * ROWS_PER_GROUP + k * 16, 16)] * 8
            la[pl.ds(k * 16, 16)] = v
            lb[pl.ds(k * 16, 16)] = v + 1

    def fire_group(g, b):
        ra, rb = rabufs[b]
        la, lb = lists[b]
        for off, n in DESCS:
            pltpu.async_copy(w_hbm.at[la.at[pl.ds(off, n)]],
                             ra.at[pl.ds(off, n)], g_isems[b])
            pltpu.async_copy(w_hbm.at[lb.at[pl.ds(off, n)]],
                             rb.at[pl.ds(off, n)], g_isems[b])

    def wait_group(g, b):
        ra, rb = rabufs[b]
        la, lb = lists[b]
        for off, n in DESCS:
            pltpu.make_async_copy(w_hbm.at[la.at[pl.ds(off, n)]],
                                  ra.at[pl.ds(off, n)], g_isems[b]).wait()
            pltpu.make_async_copy(w_hbm.at[lb.at[pl.ds(off, n)]],
                                  rb.at[pl.ds(off, n)], g_isems[b]).wait()

    def out_dst(g):
        return lu_hbm.at[pl.ds(s_base + g * GROUP, GROUP)]

    build_lists(0, 0)
    fire_group(0, 0)

    @pl.loop(0, GROUPS // 2)
    def _g(p):
        for b in range(2):
            g = 2 * p + b
            nb = 1 - b
            wait_group(g, b)
            @pl.when(g + 1 < GROUPS)
            def _():
                build_lists(g + 1, nb)
                fire_group(g + 1, nb)
            @pl.when(g >= 1)
            def _():
                pltpu.make_async_copy(obuf, out_dst(g - 1), g_osem).wait()
            _scale_relayout(rabufs[b][0], rabufs[b][1], obuf)
            pltpu.async_copy(obuf, out_dst(g), g_osem)

    pltpu.make_async_copy(obuf, out_dst(GROUPS - 1), g_osem).wait()


def _sc_gather(idx_flat, w16):
    mesh = plsc.VectorSubcoreMesh(core_axis_name="c", subcore_axis_name="s")
    k = pl.kernel(
        _gather_body,
        out_type=jax.ShapeDtypeStruct((B_SAMPLES, SEQ, D), jnp.float32),
        mesh=mesh,
        compiler_params=pltpu.CompilerParams(use_tc_tiling_on_sc=False),
        scratch_types=[
            pltpu.VMEM((IDX_PER_W,), jnp.int32),
            pltpu.VMEM((ROWS_PER_GROUP, 16), jnp.float32),
            pltpu.VMEM((ROWS_PER_GROUP, 16), jnp.float32),
            pltpu.VMEM((ROWS_PER_GROUP, 16), jnp.float32),
            pltpu.VMEM((ROWS_PER_GROUP, 16), jnp.float32),
            pltpu.VMEM((ROWS_PER_GROUP,), jnp.int32),
            pltpu.VMEM((ROWS_PER_GROUP,), jnp.int32),
            pltpu.VMEM((ROWS_PER_GROUP,), jnp.int32),
            pltpu.VMEM((ROWS_PER_GROUP,), jnp.int32),
            pltpu.VMEM((GROUP, SEQ, D), jnp.float32),
        ] + [pltpu.SemaphoreType.DMA] * 3,
    )
    return k(idx_flat, w16)


def kernel(indices, weight):
    masked, mpad = _scale_table(weight)
    w16 = mpad.reshape(N_ROWS * 8, 16)
    idx_flat = indices.astype(jnp.int32).reshape(NW, IDX_PER_W)
    lu = _sc_gather(idx_flat, w16)
    return lu, masked
